# speedup vs baseline: 3.0252x; 3.0252x over previous
import jax
import jax.numpy as jnp
from jax.experimental import pallas as pl
from jax.experimental.pallas import tpu as pltpu


def _morph_kernel(x_ref, dil_ref, ero_ref):
    x = x_ref[0]

    left = jnp.concatenate([x[:, :1], x[:, :-1]], axis=1)
    right = jnp.concatenate([x[:, 1:], x[:, -1:]], axis=1)
    hmax = jnp.maximum(x, jnp.maximum(left, right))
    hmin = jnp.minimum(x, jnp.minimum(left, right))

    up = jnp.concatenate([hmax[:1, :], hmax[:-1, :]], axis=0)
    down = jnp.concatenate([hmax[1:, :], hmax[-1:, :]], axis=0)
    dil_ref[0] = jnp.maximum(hmax, jnp.maximum(up, down))

    upn = jnp.concatenate([hmin[:1, :], hmin[:-1, :]], axis=0)
    downn = jnp.concatenate([hmin[1:, :], hmin[-1:, :]], axis=0)
    ero_ref[0] = jnp.minimum(hmin, jnp.minimum(upn, downn))


@jax.jit
def kernel(x):
    n, c, h, w = x.shape
    xf = x.reshape(n * c, h, w)
    dil, ero = pl.pallas_call(
        _morph_kernel,
        grid=(n * c,),
        in_specs=[pl.BlockSpec((1, h, w), lambda i: (i, 0, 0))],
        out_specs=[
            pl.BlockSpec((1, h, w), lambda i: (i, 0, 0)),
            pl.BlockSpec((1, h, w), lambda i: (i, 0, 0)),
        ],
        out_shape=[
            jax.ShapeDtypeStruct((n * c, h, w), x.dtype),
            jax.ShapeDtypeStruct((n * c, h, w), x.dtype),
        ],
        compiler_params=pltpu.CompilerParams(
            dimension_semantics=("parallel",),
        ),
    )(xf)
    return dil.reshape(n, c, h, w), ero.reshape(n, c, h, w)


# speedup vs baseline: 4.1540x; 1.3731x over previous
import jax
import jax.numpy as jnp
from jax.experimental import pallas as pl
from jax.experimental.pallas import tpu as pltpu


def _morph_kernel(x_ref, dil_ref, ero_ref):
    x = x_ref[...]

    left = jnp.concatenate([x[:, :, :1], x[:, :, :-1]], axis=2)
    right = jnp.concatenate([x[:, :, 1:], x[:, :, -1:]], axis=2)
    hmax = jnp.maximum(x, jnp.maximum(left, right))
    hmin = jnp.minimum(x, jnp.minimum(left, right))

    up = jnp.concatenate([hmax[:, :1, :], hmax[:, :-1, :]], axis=1)
    down = jnp.concatenate([hmax[:, 1:, :], hmax[:, -1:, :]], axis=1)
    dil_ref[...] = jnp.maximum(hmax, jnp.maximum(up, down))

    upn = jnp.concatenate([hmin[:, :1, :], hmin[:, :-1, :]], axis=1)
    downn = jnp.concatenate([hmin[:, 1:, :], hmin[:, -1:, :]], axis=1)
    ero_ref[...] = jnp.minimum(hmin, jnp.minimum(upn, downn))


@jax.jit
def kernel(x):
    n, c, h, w = x.shape
    xf = x.reshape(n * c, h, w)
    blk = 4
    dil, ero = pl.pallas_call(
        _morph_kernel,
        grid=(n * c // blk,),
        in_specs=[pl.BlockSpec((blk, h, w), lambda i: (i, 0, 0))],
        out_specs=[
            pl.BlockSpec((blk, h, w), lambda i: (i, 0, 0)),
            pl.BlockSpec((blk, h, w), lambda i: (i, 0, 0)),
        ],
        out_shape=[
            jax.ShapeDtypeStruct((n * c, h, w), x.dtype),
            jax.ShapeDtypeStruct((n * c, h, w), x.dtype),
        ],
        compiler_params=pltpu.CompilerParams(
            dimension_semantics=("parallel",),
        ),
    )(xf)
    return dil.reshape(n, c, h, w), ero.reshape(n, c, h, w)


# speedup vs baseline: 4.3273x; 1.0417x over previous
import jax
import jax.numpy as jnp
from jax.experimental import pallas as pl
from jax.experimental.pallas import tpu as pltpu


def _morph_kernel(x_ref, dil_ref, ero_ref):
    x = x_ref[...]

    left = jnp.concatenate([x[:, :, :1], x[:, :, :-1]], axis=2)
    right = jnp.concatenate([x[:, :, 1:], x[:, :, -1:]], axis=2)
    hmax = jnp.maximum(x, jnp.maximum(left, right))
    hmin = jnp.minimum(x, jnp.minimum(left, right))

    up = jnp.concatenate([hmax[:, :1, :], hmax[:, :-1, :]], axis=1)
    down = jnp.concatenate([hmax[:, 1:, :], hmax[:, -1:, :]], axis=1)
    dil_ref[...] = jnp.maximum(hmax, jnp.maximum(up, down))

    upn = jnp.concatenate([hmin[:, :1, :], hmin[:, :-1, :]], axis=1)
    downn = jnp.concatenate([hmin[:, 1:, :], hmin[:, -1:, :]], axis=1)
    ero_ref[...] = jnp.minimum(hmin, jnp.minimum(upn, downn))


@jax.jit
def kernel(x):
    n, c, h, w = x.shape
    xf = x.reshape(n * c, h, w)
    blk = 8
    dil, ero = pl.pallas_call(
        _morph_kernel,
        grid=(n * c // blk,),
        in_specs=[pl.BlockSpec((blk, h, w), lambda i: (i, 0, 0))],
        out_specs=[
            pl.BlockSpec((blk, h, w), lambda i: (i, 0, 0)),
            pl.BlockSpec((blk, h, w), lambda i: (i, 0, 0)),
        ],
        out_shape=[
            jax.ShapeDtypeStruct((n * c, h, w), x.dtype),
            jax.ShapeDtypeStruct((n * c, h, w), x.dtype),
        ],
        compiler_params=pltpu.CompilerParams(
            dimension_semantics=("parallel",),
        ),
    )(xf)
    return dil.reshape(n, c, h, w), ero.reshape(n, c, h, w)
